# baseline (speedup 1.0000x reference)
import functools

import jax
import jax.numpy as jnp
from jax import lax
from jax.experimental import pallas as pl
from jax.experimental.pallas import tpu as pltpu
from jax.experimental.pallas import tpu_sc as plsc

NUM_CORES = 2
NUM_SUBCORES = 16
NLANES = 16
NW = NUM_CORES * NUM_SUBCORES


def _sc_histogram(x_flat, B, L, V):
    rows_per_w = B // NW
    chunk_rows = 64
    n_chunks = rows_per_w // chunk_rows
    n_groups = chunk_rows // NLANES

    mesh = plsc.VectorSubcoreMesh(
        core_axis_name="c", subcore_axis_name="s",
        num_cores=NUM_CORES, num_subcores=NUM_SUBCORES)

    @functools.partial(
        pl.kernel,
        out_type=jax.ShapeDtypeStruct((B * V,), jnp.float32),
        mesh=mesh,
        compiler_params=pltpu.CompilerParams(
            needs_layout_passes=False, disable_bounds_checks=True),
        scratch_types=[
            pltpu.VMEM((chunk_rows * L,), jnp.int32),
            pltpu.VMEM((chunk_rows * L,), jnp.int32),
            pltpu.VMEM((rows_per_w * V,), jnp.float32),
            pltpu.SemaphoreType.DMA,
            pltpu.SemaphoreType.DMA,
        ],
    )
    def hist(x_hbm, out_hbm, xb0, xb1, counts, sem0, sem1):
        wid = lax.axis_index("s") * NUM_CORES + lax.axis_index("c")
        row0 = wid * rows_per_w

        lane = lax.iota(jnp.int32, NLANES)
        laneL = lane * L
        laneV = lane * V
        ones = jnp.full((NLANES,), 1.0, jnp.float32)
        zeros = jnp.zeros((NLANES,), jnp.float32)

        @plsc.parallel_loop(0, (rows_per_w * V) // NLANES, unroll=8)
        def _(j):
            counts[pl.ds(j * NLANES, NLANES)] = zeros

        xbufs = (xb0, xb1)
        sems = (sem0, sem1)

        def start_chunk(c):
            off = (row0 + c * chunk_rows) * L
            return pltpu.async_copy(
                x_hbm.at[pl.ds(off, chunk_rows * L)],
                xbufs[c % 2], sems[c % 2])

        pending = start_chunk(0)
        for c in range(n_chunks):
            pending.wait()
            if c + 1 < n_chunks:
                pending = start_chunk(c + 1)
            xb = xbufs[c % 2]
            for g in range(n_groups):
                src_base = laneL + (g * NLANES * L)
                dst_base = laneV + ((c * chunk_rows + g * NLANES) * V)

                @plsc.parallel_loop(0, L, unroll=8)
                def _(l):
                    v = plsc.load_gather(xb, [src_base + l])
                    plsc.addupdate_scatter(counts, [dst_base + v], ones)

        pltpu.sync_copy(counts,
                        out_hbm.at[pl.ds(row0 * V, rows_per_w * V)])

    return hist(x_flat)


def _tc_mlp(counts, emb, W1, b1, W2, b2, L):
    B, V = counts.shape
    E = emb.shape[1]
    H = W1.shape[1]
    P = W2.shape[1]
    blk = 2048
    inv_l = 1.0 / float(L)

    def body(c_ref, emb_ref, w1_ref, b1_ref, w2_ref, b2_ref, o_ref):
        ew = jnp.dot(emb_ref[...], w1_ref[...],
                     preferred_element_type=jnp.float32) * inv_l
        h = jnp.maximum(
            jnp.dot(c_ref[...].astype(jnp.bfloat16),
                    ew.astype(jnp.bfloat16),
                    preferred_element_type=jnp.float32)
            + b1_ref[...], 0.0)
        o = jnp.dot(h.astype(jnp.bfloat16),
                    w2_ref[...].astype(jnp.bfloat16),
                    preferred_element_type=jnp.float32) + b2_ref[...]
        ss = jnp.sum(o * o, axis=1, keepdims=True)
        o_ref[...] = o / jnp.maximum(jnp.sqrt(ss), 1e-12)

    return pl.pallas_call(
        body,
        grid=(B // blk,),
        in_specs=[
            pl.BlockSpec((blk, V), lambda i: (i, 0)),
            pl.BlockSpec((V, E), lambda i: (0, 0)),
            pl.BlockSpec((E, H), lambda i: (0, 0)),
            pl.BlockSpec((1, H), lambda i: (0, 0)),
            pl.BlockSpec((H, P), lambda i: (0, 0)),
            pl.BlockSpec((1, P), lambda i: (0, 0)),
        ],
        out_specs=pl.BlockSpec((blk, P), lambda i: (i, 0)),
        out_shape=jax.ShapeDtypeStruct((B, P), jnp.float32),
    )(counts, emb, W1, b1.reshape(1, H), W2, b2.reshape(1, P))


def kernel(x, emb, W1, b1, W2, b2):
    B, L = x.shape
    V = emb.shape[0]
    h = B // 2
    assert h % (NW * NLANES) == 0
    ca = _sc_histogram(x[:h].reshape(-1), h, L, V).reshape(h, V)
    cb = _sc_histogram(x[h:].reshape(-1), h, L, V).reshape(h, V)
    oa = _tc_mlp(ca, emb, W1, b1, W2, b2, L)
    ob = _tc_mlp(cb, emb, W1, b1, W2, b2, L)
    return jnp.concatenate([oa, ob], axis=0)


# speedup vs baseline: 1.0502x; 1.0502x over previous
import functools

import jax
import jax.numpy as jnp
from jax import lax
from jax.experimental import pallas as pl
from jax.experimental.pallas import tpu as pltpu
from jax.experimental.pallas import tpu_sc as plsc

NUM_CORES = 2
NUM_SUBCORES = 16
NLANES = 16
NW = NUM_CORES * NUM_SUBCORES


def _sc_histogram(x_flat, B, L, V):
    rows_per_w = B // NW
    chunk_rows = 64
    n_chunks = rows_per_w // chunk_rows
    n_groups = chunk_rows // NLANES

    mesh = plsc.VectorSubcoreMesh(
        core_axis_name="c", subcore_axis_name="s",
        num_cores=NUM_CORES, num_subcores=NUM_SUBCORES)

    @functools.partial(
        pl.kernel,
        out_type=jax.ShapeDtypeStruct((B * V,), jnp.float32),
        mesh=mesh,
        compiler_params=pltpu.CompilerParams(
            needs_layout_passes=False, disable_bounds_checks=True,
            use_tc_tiling_on_sc=False),
        scratch_types=[
            pltpu.VMEM((chunk_rows * L // 128, 128), jnp.int32),
            pltpu.VMEM((chunk_rows * L // 128, 128), jnp.int32),
            pltpu.VMEM((rows_per_w * V,), jnp.float32),
            pltpu.SemaphoreType.DMA,
            pltpu.SemaphoreType.DMA,
        ],
    )
    def hist(x_hbm, out_hbm, xb0, xb1, counts, sem0, sem1):
        wid = lax.axis_index("s") * NUM_CORES + lax.axis_index("c")
        row0 = wid * rows_per_w

        lane = lax.iota(jnp.int32, NLANES)
        laneL = lane * L
        laneV = lane * V
        ones = jnp.full((NLANES,), 1.0, jnp.float32)
        zeros = jnp.zeros((NLANES,), jnp.float32)

        @plsc.parallel_loop(0, (rows_per_w * V) // NLANES, unroll=8)
        def _(j):
            counts[pl.ds(j * NLANES, NLANES)] = zeros

        xbufs = (xb0, xb1)
        sems = (sem0, sem1)

        chunk_words = chunk_rows * L

        def start_chunk(c):
            off = (row0 + c * chunk_rows) * L // 128
            return pltpu.async_copy(
                x_hbm.at[pl.ds(off, chunk_words // 128), :],
                xbufs[c % 2], sems[c % 2])

        pending = start_chunk(0)
        for c in range(n_chunks):
            pending.wait()
            if c + 1 < n_chunks:
                pending = start_chunk(c + 1)
            xb = xbufs[c % 2]
            for g in range(n_groups):
                src_base = laneL + (g * NLANES * L)
                dst_base = laneV + ((c * chunk_rows + g * NLANES) * V)

                @plsc.parallel_loop(0, L, unroll=8)
                def _(l):
                    idx = src_base + l
                    v = plsc.load_gather(xb, [idx >> 7, idx & 127])
                    plsc.addupdate_scatter(counts, [dst_base + v], ones)

        pltpu.sync_copy(counts,
                        out_hbm.at[pl.ds(row0 * V, rows_per_w * V)])

    return hist(x_flat)


def _tc_mlp(counts, emb, W1, b1, W2, b2, L):
    B, V = counts.shape
    E = emb.shape[1]
    H = W1.shape[1]
    P = W2.shape[1]
    blk = 2048
    inv_l = 1.0 / float(L)

    def body(c_ref, emb_ref, w1_ref, b1_ref, w2_ref, b2_ref, o_ref):
        ew = jnp.dot(emb_ref[...], w1_ref[...],
                     preferred_element_type=jnp.float32) * inv_l
        h = jnp.maximum(
            jnp.dot(c_ref[...].astype(jnp.bfloat16),
                    ew.astype(jnp.bfloat16),
                    preferred_element_type=jnp.float32)
            + b1_ref[...], 0.0)
        o = jnp.dot(h.astype(jnp.bfloat16),
                    w2_ref[...].astype(jnp.bfloat16),
                    preferred_element_type=jnp.float32) + b2_ref[...]
        ss = jnp.sum(o * o, axis=1, keepdims=True)
        o_ref[...] = o / jnp.maximum(jnp.sqrt(ss), 1e-12)

    return pl.pallas_call(
        body,
        grid=(B // blk,),
        in_specs=[
            pl.BlockSpec((blk, V), lambda i: (i, 0)),
            pl.BlockSpec((V, E), lambda i: (0, 0)),
            pl.BlockSpec((E, H), lambda i: (0, 0)),
            pl.BlockSpec((1, H), lambda i: (0, 0)),
            pl.BlockSpec((H, P), lambda i: (0, 0)),
            pl.BlockSpec((1, P), lambda i: (0, 0)),
        ],
        out_specs=pl.BlockSpec((blk, P), lambda i: (i, 0)),
        out_shape=jax.ShapeDtypeStruct((B, P), jnp.float32),
    )(counts, emb, W1, b1.reshape(1, H), W2, b2.reshape(1, P))


def kernel(x, emb, W1, b1, W2, b2):
    B, L = x.shape
    V = emb.shape[0]
    assert B % (NW * NLANES) == 0 and (B * L) % 128 == 0
    counts = _sc_histogram(
        x.reshape(B * L // 128, 128), B, L, V).reshape(B, V)
    return _tc_mlp(counts, emb, W1, b1, W2, b2, L)


# speedup vs baseline: 1.0513x; 1.0011x over previous
import functools

import jax
import jax.numpy as jnp
from jax import lax
from jax.experimental import pallas as pl
from jax.experimental.pallas import tpu as pltpu
from jax.experimental.pallas import tpu_sc as plsc

NUM_CORES = 2
NUM_SUBCORES = 16
NLANES = 16
NW = NUM_CORES * NUM_SUBCORES


def _sc_histogram(x_flat, B, L, V):
    rows_per_w = B // NW
    chunk_rows = 64
    n_chunks = rows_per_w // chunk_rows
    n_groups = chunk_rows // NLANES

    mesh = plsc.VectorSubcoreMesh(
        core_axis_name="c", subcore_axis_name="s",
        num_cores=NUM_CORES, num_subcores=NUM_SUBCORES)

    @functools.partial(
        pl.kernel,
        out_type=jax.ShapeDtypeStruct((B * V,), jnp.float32),
        mesh=mesh,
        compiler_params=pltpu.CompilerParams(
            needs_layout_passes=False, disable_bounds_checks=True),
        scratch_types=[
            pltpu.VMEM((chunk_rows * L,), jnp.int32),
            pltpu.VMEM((chunk_rows * L,), jnp.int32),
            pltpu.VMEM((rows_per_w * V,), jnp.float32),
            pltpu.SemaphoreType.DMA,
            pltpu.SemaphoreType.DMA,
        ],
    )
    def hist(x_hbm, out_hbm, xb0, xb1, counts, sem0, sem1):
        wid = lax.axis_index("s") * NUM_CORES + lax.axis_index("c")
        row0 = wid * rows_per_w

        lane = lax.iota(jnp.int32, NLANES)
        laneL = lane * L
        laneV = lane * V
        ones = jnp.full((NLANES,), 1.0, jnp.float32)
        zeros = jnp.zeros((NLANES,), jnp.float32)

        @plsc.parallel_loop(0, (rows_per_w * V) // NLANES, unroll=8)
        def _(j):
            counts[pl.ds(j * NLANES, NLANES)] = zeros

        xbufs = (xb0, xb1)
        sems = (sem0, sem1)

        def start_chunk(c):
            off = (row0 + c * chunk_rows) * L
            return pltpu.async_copy(
                x_hbm.at[pl.ds(off, chunk_rows * L)],
                xbufs[c % 2], sems[c % 2])

        pending = start_chunk(0)
        for c in range(n_chunks):
            pending.wait()
            if c + 1 < n_chunks:
                pending = start_chunk(c + 1)
            xb = xbufs[c % 2]
            for g in range(n_groups):
                src_base = laneL + (g * NLANES * L)
                dst_base = laneV + ((c * chunk_rows + g * NLANES) * V)

                @plsc.parallel_loop(0, L, unroll=8)
                def _(l):
                    v = plsc.load_gather(xb, [src_base + l])
                    plsc.addupdate_scatter(counts, [dst_base + v], ones)

        pltpu.sync_copy(counts,
                        out_hbm.at[pl.ds(row0 * V, rows_per_w * V)])

    return hist(x_flat)


def _tc_mlp(counts, emb, W1, b1, W2, b2, L):
    B, V = counts.shape
    E = emb.shape[1]
    H = W1.shape[1]
    P = W2.shape[1]
    blk = 2048
    inv_l = 1.0 / float(L)

    def body(c_ref, emb_ref, w1_ref, b1_ref, w2_ref, b2_ref, o_ref):
        ew = jnp.dot(emb_ref[...], w1_ref[...],
                     preferred_element_type=jnp.float32) * inv_l
        h = jnp.maximum(
            jnp.dot(c_ref[...].astype(jnp.bfloat16),
                    ew.astype(jnp.bfloat16),
                    preferred_element_type=jnp.float32)
            + b1_ref[...], 0.0)
        o = jnp.dot(h.astype(jnp.bfloat16),
                    w2_ref[...].astype(jnp.bfloat16),
                    preferred_element_type=jnp.float32) + b2_ref[...]
        ss = jnp.sum(o * o, axis=1, keepdims=True)
        o_ref[...] = o / jnp.maximum(jnp.sqrt(ss), 1e-12)

    return pl.pallas_call(
        body,
        grid=(B // blk,),
        in_specs=[
            pl.BlockSpec((blk, V), lambda i: (i, 0)),
            pl.BlockSpec((V, E), lambda i: (0, 0)),
            pl.BlockSpec((E, H), lambda i: (0, 0)),
            pl.BlockSpec((1, H), lambda i: (0, 0)),
            pl.BlockSpec((H, P), lambda i: (0, 0)),
            pl.BlockSpec((1, P), lambda i: (0, 0)),
        ],
        out_specs=pl.BlockSpec((blk, P), lambda i: (i, 0)),
        out_shape=jax.ShapeDtypeStruct((B, P), jnp.float32),
    )(counts, emb, W1, b1.reshape(1, H), W2, b2.reshape(1, P))


def kernel(x, emb, W1, b1, W2, b2):
    B, L = x.shape
    V = emb.shape[0]
    assert B % (NW * NLANES) == 0
    counts = _sc_histogram(x.reshape(-1), B, L, V).reshape(B, V)
    return _tc_mlp(counts, emb, W1, b1, W2, b2, L)
